# Initial kernel scaffold; baseline (speedup 1.0000x reference)
import jax
import jax.numpy as jnp
from jax import lax
from jax.experimental import pallas as pl
from jax.experimental.pallas import tpu as pltpu
from jax.experimental.pallas import tpu_sc as plsc

B = 1024
M = 50
S = 20
V = 100000
E = 64
L = 16

CH = 16
G = 4
GS = CH * S // G


def _sc_pool_body(story_idx, q_idx, e0, e1, e2, e3,
                  m0, m1, m2, m3, qsum,
                  idx_v, rows_v, out_v, sem):
    wid = lax.axis_index("s") * 2 + lax.axis_index("c")

    def pool_chunk(table, out_hbm, seg_base):
        handles = [
            pltpu.async_copy(table.at[idx_v.at[g]],
                             rows_v.at[pl.ds(g * GS, GS)], sem)
            for g in range(G)
        ]
        for h in handles:
            h.wait()

        def seg_body(c, carry):
            base = c * S
            for cg in range(E // L):
                sl = pl.ds(cg * L, L)
                acc = rows_v[base, sl]
                for s2 in range(1, S):
                    acc = acc + rows_v[base + s2, sl]
            out_v[c, sl] = acc
            return carry

        lax.fori_loop(0, CH, seg_body, 0)
        pltpu.sync_copy(out_v, out_hbm.at[pl.ds(seg_base, CH)])

    n_steps = story_idx.shape[1]

    def step_body(st, carry):
        pltpu.sync_copy(story_idx.at[wid, st], idx_v)
        seg_base = (wid * n_steps + st) * CH
        pool_chunk(e0, m0, seg_base)
        pool_chunk(e1, m1, seg_base)
        pool_chunk(e2, m2, seg_base)
        pool_chunk(e3, m3, seg_base)
        return carry

    lax.fori_loop(0, n_steps, step_body, 0)

    nq_steps = q_idx.shape[1]

    def qstep_body(st, carry):
        pltpu.sync_copy(q_idx.at[wid, st], idx_v)
        seg_base = (wid * nq_steps + st) * CH
        pool_chunk(e0, qsum, seg_base)
        return carry

    lax.fori_loop(0, nq_steps, qstep_body, 0)


def _hops_body(q_ref, m0_ref, m1_ref, m2_ref, m3_ref, u_ref):
    u = q_ref[...]
    ms = (m0_ref[...], m1_ref[...], m2_ref[...], m3_ref[...])
    for i in range(3):
        m, c = ms[i], ms[i + 1]
        lg = lax.dot_general(m, u, (((2,), (1,)), ((0,), (0,))),
                             preferred_element_type=jnp.float32)
        lg = lg - jnp.max(lg, axis=1, keepdims=True)
        ex = jnp.exp(lg)
        p = ex / jnp.sum(ex, axis=1, keepdims=True)
        u = lax.dot_general(p, c, (((1,), (1,)), ((0,), (0,))),
                            preferred_element_type=jnp.float32) + u
    u_ref[...] = u


BT = 256
VT = 5000
NB = B // BT
NV = V // VT


def _sumexp_body(u_ref, e_ref, s_ref, acc_ref):
    v = pl.program_id(0)
    b = pl.program_id(1)
    lg = lax.dot_general(u_ref[...], e_ref[...], (((1,), (1,)), ((), ())),
                         preferred_element_type=jnp.float32)
    part = jnp.sum(jnp.exp(lg), axis=1)[None, :]

    @pl.when(v == 0)
    def _():
        acc_ref[pl.ds(b, 1), :] = part

    @pl.when(v != 0)
    def _():
        acc_ref[pl.ds(b, 1), :] = acc_ref[pl.ds(b, 1), :] + part

    s_ref[...] = acc_ref[pl.ds(b, 1), :]


def _normexp_body(u_ref, e_ref, s_ref, o_ref):
    lg = lax.dot_general(u_ref[...], e_ref[...], (((1,), (1,)), ((), ())),
                         preferred_element_type=jnp.float32)
    o_ref[...] = jnp.exp(lg) * (1.0 / s_ref[0, :])[:, None]


def kernel(story, question, emb_0, emb_1, emb_2, emb_3):
    story = story.astype(jnp.int32)
    question = question.astype(jnp.int32)

    info = plsc.get_sparse_core_info()
    nw = info.num_cores * info.num_subcores

    n_steps = (B * M) // (nw * CH)
    nq_steps = B // (nw * CH)
    story_idx = story.reshape(nw, n_steps, G, GS)
    q_idx = question.reshape(nw, nq_steps, G, GS)

    mesh = plsc.VectorSubcoreMesh(core_axis_name="c", subcore_axis_name="s")
    pooled = pl.kernel(
        _sc_pool_body,
        out_type=(
            jax.ShapeDtypeStruct((B * M, E), jnp.float32),
            jax.ShapeDtypeStruct((B * M, E), jnp.float32),
            jax.ShapeDtypeStruct((B * M, E), jnp.float32),
            jax.ShapeDtypeStruct((B * M, E), jnp.float32),
            jax.ShapeDtypeStruct((B, E), jnp.float32),
        ),
        mesh=mesh,
        scratch_types=[
            pltpu.VMEM((G, GS), jnp.int32),
            pltpu.VMEM((CH * S, E), jnp.float32),
            pltpu.VMEM((CH, E), jnp.float32),
            pltpu.SemaphoreType.DMA,
        ],
    )(story_idx, q_idx, emb_0, emb_1, emb_2, emb_3)
    m0, m1, m2, m3, qsum = pooled
    m0 = m0.reshape(B, M, E)
    m1 = m1.reshape(B, M, E)
    m2 = m2.reshape(B, M, E)
    m3 = m3.reshape(B, M, E)

    bt_h = 128
    u = pl.pallas_call(
        _hops_body,
        grid=(B // bt_h,),
        in_specs=[
            pl.BlockSpec((bt_h, E), lambda i: (i, 0)),
            pl.BlockSpec((bt_h, M, E), lambda i: (i, 0, 0)),
            pl.BlockSpec((bt_h, M, E), lambda i: (i, 0, 0)),
            pl.BlockSpec((bt_h, M, E), lambda i: (i, 0, 0)),
            pl.BlockSpec((bt_h, M, E), lambda i: (i, 0, 0)),
        ],
        out_specs=pl.BlockSpec((bt_h, E), lambda i: (i, 0)),
        out_shape=jax.ShapeDtypeStruct((B, E), jnp.float32),
    )(qsum, m0, m1, m2, m3)

    s = pl.pallas_call(
        _sumexp_body,
        grid=(NV, NB),
        in_specs=[
            pl.BlockSpec((BT, E), lambda v, b: (b, 0)),
            pl.BlockSpec((VT, E), lambda v, b: (v, 0)),
        ],
        out_specs=pl.BlockSpec((1, BT), lambda v, b: (b, 0)),
        out_shape=jax.ShapeDtypeStruct((NB, BT), jnp.float32),
        scratch_shapes=[pltpu.VMEM((NB, BT), jnp.float32)],
    )(u, emb_3)

    out = pl.pallas_call(
        _normexp_body,
        grid=(NV, NB),
        in_specs=[
            pl.BlockSpec((BT, E), lambda v, b: (b, 0)),
            pl.BlockSpec((VT, E), lambda v, b: (v, 0)),
            pl.BlockSpec((1, BT), lambda v, b: (b, 0)),
        ],
        out_specs=pl.BlockSpec((BT, VT), lambda v, b: (b, v)),
        out_shape=jax.ShapeDtypeStruct((B, V), jnp.float32),
    )(u, emb_3, s)
    return out


# speedup vs baseline: 7.0150x; 7.0150x over previous
import jax
import jax.numpy as jnp
from jax import lax
from jax.experimental import pallas as pl
from jax.experimental.pallas import tpu as pltpu
from jax.experimental.pallas import tpu_sc as plsc

B = 1024
M = 50
S = 20
V = 100000
E = 64
EA = 4 * E
L = 16

CH = 8
G = 2
GS = CH * S // G


def _sc_pool_body(story_idx, q_idx, tab,
                  msum, qsum,
                  idx_v, rows_v, out_v, sem):
    wid = lax.axis_index("s") * 2 + lax.axis_index("c")

    def pool_chunk(out_hbm, seg_base):
        handles = [
            pltpu.async_copy(tab.at[idx_v.at[g]],
                             rows_v.at[pl.ds(g * GS, GS)], sem)
            for g in range(G)
        ]
        for h in handles:
            h.wait()

        def seg_body(c, carry):
            base = c * S
            for cg in range(EA // L):
                sl = pl.ds(cg * L, L)
                acc = rows_v[base, sl]
                for s2 in range(1, S):
                    acc = acc + rows_v[base + s2, sl]
                out_v[c, sl] = acc
            return carry

        lax.fori_loop(0, CH, seg_body, 0)
        pltpu.sync_copy(out_v, out_hbm.at[pl.ds(seg_base, CH)])

    n_steps = story_idx.shape[1]

    def step_body(st, carry):
        pltpu.sync_copy(story_idx.at[wid, st], idx_v)
        pool_chunk(msum, (wid * n_steps + st) * CH)
        return carry

    lax.fori_loop(0, n_steps, step_body, 0)

    nq_steps = q_idx.shape[1]

    def qstep_body(st, carry):
        pltpu.sync_copy(q_idx.at[wid, st], idx_v)
        pool_chunk(qsum, (wid * nq_steps + st) * CH)
        return carry

    lax.fori_loop(0, nq_steps, qstep_body, 0)


def _hops_body(q_ref, mall_ref, u_ref):
    u = q_ref[...][:, :E]
    mall = mall_ref[...]
    ms = tuple(mall[:, :, i * E:(i + 1) * E] for i in range(4))
    for i in range(3):
        m, c = ms[i], ms[i + 1]
        lg = lax.dot_general(m, u, (((2,), (1,)), ((0,), (0,))),
                             preferred_element_type=jnp.float32)
        lg = lg - jnp.max(lg, axis=1, keepdims=True)
        ex = jnp.exp(lg)
        p = ex / jnp.sum(ex, axis=1, keepdims=True)
        u = lax.dot_general(p, c, (((1,), (1,)), ((0,), (0,))),
                            preferred_element_type=jnp.float32) + u
    u_ref[...] = u


BT = 256
VT = 5000
NB = B // BT
NV = V // VT


def _sumexp_body(u_ref, e_ref, s_ref):
    v = pl.program_id(0)
    b = pl.program_id(1)
    lg = lax.dot_general(u_ref[...], e_ref[...], (((1,), (1,)), ((), ())),
                         preferred_element_type=jnp.float32)
    part = jnp.sum(jnp.exp(lg), axis=1)[None, :]

    @pl.when(v == 0)
    def _():
        s_ref[pl.ds(b, 1), :] = part

    @pl.when(v != 0)
    def _():
        s_ref[pl.ds(b, 1), :] = s_ref[pl.ds(b, 1), :] + part


VT2 = 6272
NV2 = 15
VTAIL = V - NV2 * VT2


def _normexp_body(u_ref, e_ref, s_ref, o_hbm, buf_ref, sem):
    v = pl.program_id(0)
    b = pl.program_id(1)
    step = v * NB + b
    slot = step % 2
    lg = lax.dot_general(u_ref[...], e_ref[...], (((1,), (1,)), ((), ())),
                         preferred_element_type=jnp.float32)
    sc = s_ref[pl.ds(b, 1), :]
    res = jnp.exp(lg) * (1.0 / sc[0, :])[:, None]

    dst = o_hbm.at[pl.ds(b * BT, BT), pl.ds(v * VT2, VT2)]

    @pl.when(step >= 2)
    def _():
        pltpu.make_async_copy(buf_ref.at[slot], dst, sem).wait()

    buf_ref[pl.ds(slot, 1), :, :] = res[None]
    pltpu.make_async_copy(buf_ref.at[slot], dst, sem).start()

    @pl.when(step == NV2 * NB - 1)
    def _():
        pltpu.make_async_copy(buf_ref.at[slot], dst, sem).wait()
        pltpu.make_async_copy(buf_ref.at[1 - slot], dst, sem).wait()


def _tail_body(o_in, u_ref, e_ref, s_ref, o_hbm, buf_ref, sem):
    del o_in
    b = pl.program_id(0)
    lg = lax.dot_general(u_ref[...], e_ref[...], (((1,), (1,)), ((), ())),
                         preferred_element_type=jnp.float32)
    sc = s_ref[pl.ds(b, 1), :]
    buf_ref[...] = jnp.exp(lg) * (1.0 / sc[0, :])[:, None]
    dst = o_hbm.at[pl.ds(b * BT, BT), pl.ds(NV2 * VT2, VTAIL)]
    pltpu.make_async_copy(buf_ref, dst, sem).start()
    pltpu.make_async_copy(buf_ref, dst, sem).wait()


def kernel(story, question, emb_0, emb_1, emb_2, emb_3):
    story = story.astype(jnp.int32)
    question = question.astype(jnp.int32)
    tab = jnp.concatenate([emb_0, emb_1, emb_2, emb_3], axis=1)

    info = plsc.get_sparse_core_info()
    nw = info.num_cores * info.num_subcores

    n_steps = (B * M) // (nw * CH)
    nq_steps = B // (nw * CH)
    story_idx = story.reshape(nw, n_steps, G, GS)
    q_idx = question.reshape(nw, nq_steps, G, GS)

    mesh = plsc.VectorSubcoreMesh(core_axis_name="c", subcore_axis_name="s")
    msum, qsum = pl.kernel(
        _sc_pool_body,
        out_type=(
            jax.ShapeDtypeStruct((B * M, EA), jnp.float32),
            jax.ShapeDtypeStruct((B, EA), jnp.float32),
        ),
        mesh=mesh,
        scratch_types=[
            pltpu.VMEM((G, GS), jnp.int32),
            pltpu.VMEM((CH * S, EA), jnp.float32),
            pltpu.VMEM((CH, EA), jnp.float32),
            pltpu.SemaphoreType.DMA,
        ],
    )(story_idx, q_idx, tab)
    mall = msum.reshape(B, M, EA)

    bt_h = 128
    u = pl.pallas_call(
        _hops_body,
        grid=(B // bt_h,),
        in_specs=[
            pl.BlockSpec((bt_h, EA), lambda i: (i, 0)),
            pl.BlockSpec((bt_h, M, EA), lambda i: (i, 0, 0)),
        ],
        out_specs=pl.BlockSpec((bt_h, E), lambda i: (i, 0)),
        out_shape=jax.ShapeDtypeStruct((B, E), jnp.float32),
    )(qsum, mall)

    s = pl.pallas_call(
        _sumexp_body,
        grid=(NV, NB),
        in_specs=[
            pl.BlockSpec((BT, E), lambda v, b: (b, 0)),
            pl.BlockSpec((VT, E), lambda v, b: (v, 0)),
        ],
        out_specs=pl.BlockSpec((NB, BT), lambda v, b: (0, 0)),
        out_shape=jax.ShapeDtypeStruct((NB, BT), jnp.float32),
    )(u, emb_3)

    out_main = pl.pallas_call(
        _normexp_body,
        grid=(NV2, NB),
        in_specs=[
            pl.BlockSpec((BT, E), lambda v, b: (b, 0)),
            pl.BlockSpec((VT2, E), lambda v, b: (v, 0)),
            pl.BlockSpec((NB, BT), lambda v, b: (0, 0)),
        ],
        out_specs=pl.BlockSpec(memory_space=pl.ANY),
        out_shape=jax.ShapeDtypeStruct((B, V), jnp.float32),
        scratch_shapes=[
            pltpu.VMEM((2, BT, VT2), jnp.float32),
            pltpu.SemaphoreType.DMA,
        ],
    )(u, emb_3, s)

    emb_3_tail = lax.slice(emb_3, (NV2 * VT2, 0), (V, E))
    out = pl.pallas_call(
        _tail_body,
        grid=(NB,),
        in_specs=[
            pl.BlockSpec(memory_space=pl.ANY),
            pl.BlockSpec((BT, E), lambda b: (b, 0)),
            pl.BlockSpec((VTAIL, E), lambda b: (0, 0)),
            pl.BlockSpec((NB, BT), lambda b: (0, 0)),
        ],
        out_specs=pl.BlockSpec(memory_space=pl.ANY),
        out_shape=jax.ShapeDtypeStruct((B, V), jnp.float32),
        scratch_shapes=[
            pltpu.VMEM((BT, VTAIL), jnp.float32),
            pltpu.SemaphoreType.DMA,
        ],
        input_output_aliases={0: 0},
    )(out_main, u, emb_3_tail, s)
    return out


# speedup vs baseline: 8.3245x; 1.1867x over previous
import jax
import jax.numpy as jnp
from jax import lax
from jax.experimental import pallas as pl
from jax.experimental.pallas import tpu as pltpu
from jax.experimental.pallas import tpu_sc as plsc

B = 1024
M = 50
S = 20
V = 100000
E = 64
EA = 4 * E
L = 16

CH = 8
G = 2
GS = CH * S // G


def _sc_pool_body(story_idx, q_idx, tab,
                  msum, qsum,
                  idx_v, rows_v, out_v,
                  gsem0, gsem1, ssem0, ssem1):
    wid = lax.axis_index("s") * 2 + lax.axis_index("c")
    gsems = (gsem0, gsem1)
    ssems = (ssem0, ssem1)

    def fire(idx_hbm, st, slot):
        pltpu.sync_copy(idx_hbm.at[wid, st], idx_v.at[slot])
        for g in range(G):
            pltpu.async_copy(tab.at[idx_v.at[slot, g]],
                             rows_v.at[slot, pl.ds(g * GS, GS)],
                             gsems[slot])

    def drain_gather(slot):
        for g in range(G):
            pltpu.make_async_copy(tab.at[pl.ds(0, GS)],
                                  rows_v.at[slot, pl.ds(g * GS, GS)],
                                  gsems[slot]).wait()

    def accum_store(out_hbm, seg_base, slot, drain_prev):
        dst = out_hbm.at[pl.ds(seg_base, CH)]
        @pl.when(drain_prev)
        def _():
            pltpu.make_async_copy(out_v.at[slot], dst, ssems[slot]).wait()

        def seg_body(c, carry):
            base = c * S
            for cg in range(EA // L):
                sl = pl.ds(cg * L, L)
                acc = rows_v[slot, base, sl]
                for s2 in range(1, S):
                    acc = acc + rows_v[slot, base + s2, sl]
                out_v[slot, c, sl] = acc
            return carry

        lax.fori_loop(0, CH, seg_body, 0)
        pltpu.async_copy(out_v.at[slot], dst, ssems[slot])

    n_steps = story_idx.shape[1]
    nq_steps = q_idx.shape[1]

    fire(story_idx, 0, 0)

    def outer_body(st0, carry):
        @pl.when(st0 + 1 < n_steps)
        def _():
            fire(story_idx, st0 + 1, 1)

        drain_gather(0)
        accum_store(msum, (wid * n_steps + st0) * CH, 0, st0 >= 2)

        @pl.when(st0 + 2 < n_steps)
        def _():
            fire(story_idx, st0 + 2, 0)

        drain_gather(1)
        accum_store(msum, (wid * n_steps + st0 + 1) * CH, 1, st0 >= 2)
        return carry

    lax.fori_loop(0, n_steps // 2, lambda i, c: outer_body(i * 2, c), 0)

    def qstep_body(st, carry):
        fire(q_idx, st, 0)
        drain_gather(0)
        accum_store(qsum, (wid * nq_steps + st) * CH, 0, st >= 0)
        return carry

    lax.fori_loop(0, nq_steps, qstep_body, 0)

    pltpu.make_async_copy(out_v.at[0], qsum.at[pl.ds(0, CH)], ssems[0]).wait()
    pltpu.make_async_copy(out_v.at[1], msum.at[pl.ds(0, CH)], ssems[1]).wait()


def _hops_body(q_ref, mall_ref, u_ref):
    u = q_ref[...][:, :E]
    mall = mall_ref[...]
    ms = tuple(mall[:, :, i * E:(i + 1) * E] for i in range(4))
    for i in range(3):
        m, c = ms[i], ms[i + 1]
        lg = lax.dot_general(m, u, (((2,), (1,)), ((0,), (0,))),
                             preferred_element_type=jnp.float32)
        lg = lg - jnp.max(lg, axis=1, keepdims=True)
        ex = jnp.exp(lg)
        p = ex / jnp.sum(ex, axis=1, keepdims=True)
        u = lax.dot_general(p, c, (((1,), (1,)), ((0,), (0,))),
                            preferred_element_type=jnp.float32) + u
    u_ref[...] = u


BT = 256
VT = 5000
NB = B // BT
NV = V // VT


def _sumexp_body(u_ref, e_ref, s_ref):
    v = pl.program_id(0)
    b = pl.program_id(1)
    lg = lax.dot_general(u_ref[...], e_ref[...], (((1,), (1,)), ((), ())),
                         preferred_element_type=jnp.float32)
    part = jnp.sum(jnp.exp(lg), axis=1)[None, :]

    @pl.when(v == 0)
    def _():
        s_ref[pl.ds(b, 1), :] = part

    @pl.when(v != 0)
    def _():
        s_ref[pl.ds(b, 1), :] = s_ref[pl.ds(b, 1), :] + part


VT2 = 6272
NV2 = 15
VTAIL = V - NV2 * VT2


def _normexp_body(u_ref, e_ref, s_ref, o_hbm, buf_ref, sem):
    v = pl.program_id(0)
    b = pl.program_id(1)
    step = v * NB + b
    slot = step % 2
    lg = lax.dot_general(u_ref[...], e_ref[...], (((1,), (1,)), ((), ())),
                         preferred_element_type=jnp.float32)
    sc = s_ref[pl.ds(b, 1), :]
    res = jnp.exp(lg) * (1.0 / sc[0, :])[:, None]

    dst = o_hbm.at[pl.ds(b * BT, BT), pl.ds(v * VT2, VT2)]

    @pl.when(step >= 2)
    def _():
        pltpu.make_async_copy(buf_ref.at[slot], dst, sem).wait()

    buf_ref[pl.ds(slot, 1), :, :] = res[None]
    pltpu.make_async_copy(buf_ref.at[slot], dst, sem).start()

    @pl.when(step == NV2 * NB - 1)
    def _():
        pltpu.make_async_copy(buf_ref.at[slot], dst, sem).wait()
        pltpu.make_async_copy(buf_ref.at[1 - slot], dst, sem).wait()


def _tail_body(o_in, u_ref, e_ref, s_ref, o_hbm, buf_ref, sem):
    del o_in
    b = pl.program_id(0)
    lg = lax.dot_general(u_ref[...], e_ref[...], (((1,), (1,)), ((), ())),
                         preferred_element_type=jnp.float32)
    sc = s_ref[pl.ds(b, 1), :]
    buf_ref[...] = jnp.exp(lg) * (1.0 / sc[0, :])[:, None]
    dst = o_hbm.at[pl.ds(b * BT, BT), pl.ds(NV2 * VT2, VTAIL)]
    pltpu.make_async_copy(buf_ref, dst, sem).start()
    pltpu.make_async_copy(buf_ref, dst, sem).wait()


def kernel(story, question, emb_0, emb_1, emb_2, emb_3):
    story = story.astype(jnp.int32)
    question = question.astype(jnp.int32)
    tab = jnp.concatenate([emb_0, emb_1, emb_2, emb_3], axis=1)

    info = plsc.get_sparse_core_info()
    nw = info.num_cores * info.num_subcores

    n_steps = (B * M) // (nw * CH)
    nq_steps = B // (nw * CH)
    story_idx = story.reshape(nw, n_steps, G, GS)
    q_idx = question.reshape(nw, nq_steps, G, GS)

    mesh = plsc.VectorSubcoreMesh(core_axis_name="c", subcore_axis_name="s")
    msum, qsum = pl.kernel(
        _sc_pool_body,
        out_type=(
            jax.ShapeDtypeStruct((B * M, EA), jnp.float32),
            jax.ShapeDtypeStruct((B, EA), jnp.float32),
        ),
        mesh=mesh,
        scratch_types=[
            pltpu.VMEM((2, G, GS), jnp.int32),
            pltpu.VMEM((2, CH * S, EA), jnp.float32),
            pltpu.VMEM((2, CH, EA), jnp.float32),
            pltpu.SemaphoreType.DMA,
            pltpu.SemaphoreType.DMA,
            pltpu.SemaphoreType.DMA,
            pltpu.SemaphoreType.DMA,
        ],
    )(story_idx, q_idx, tab)
    mall = msum.reshape(B, M, EA)

    bt_h = 128
    u = pl.pallas_call(
        _hops_body,
        grid=(B // bt_h,),
        in_specs=[
            pl.BlockSpec((bt_h, EA), lambda i: (i, 0)),
            pl.BlockSpec((bt_h, M, EA), lambda i: (i, 0, 0)),
        ],
        out_specs=pl.BlockSpec((bt_h, E), lambda i: (i, 0)),
        out_shape=jax.ShapeDtypeStruct((B, E), jnp.float32),
    )(qsum, mall)

    s = pl.pallas_call(
        _sumexp_body,
        grid=(NV, NB),
        in_specs=[
            pl.BlockSpec((BT, E), lambda v, b: (b, 0)),
            pl.BlockSpec((VT, E), lambda v, b: (v, 0)),
        ],
        out_specs=pl.BlockSpec((NB, BT), lambda v, b: (0, 0)),
        out_shape=jax.ShapeDtypeStruct((NB, BT), jnp.float32),
    )(u, emb_3)

    out_main = pl.pallas_call(
        _normexp_body,
        grid=(NV2, NB),
        in_specs=[
            pl.BlockSpec((BT, E), lambda v, b: (b, 0)),
            pl.BlockSpec((VT2, E), lambda v, b: (v, 0)),
            pl.BlockSpec((NB, BT), lambda v, b: (0, 0)),
        ],
        out_specs=pl.BlockSpec(memory_space=pl.ANY),
        out_shape=jax.ShapeDtypeStruct((B, V), jnp.float32),
        scratch_shapes=[
            pltpu.VMEM((2, BT, VT2), jnp.float32),
            pltpu.SemaphoreType.DMA,
        ],
    )(u, emb_3, s)

    emb_3_tail = lax.slice(emb_3, (NV2 * VT2, 0), (V, E))
    out = pl.pallas_call(
        _tail_body,
        grid=(NB,),
        in_specs=[
            pl.BlockSpec(memory_space=pl.ANY),
            pl.BlockSpec((BT, E), lambda b: (b, 0)),
            pl.BlockSpec((VTAIL, E), lambda b: (0, 0)),
            pl.BlockSpec((NB, BT), lambda b: (0, 0)),
        ],
        out_specs=pl.BlockSpec(memory_space=pl.ANY),
        out_shape=jax.ShapeDtypeStruct((B, V), jnp.float32),
        scratch_shapes=[
            pltpu.VMEM((BT, VTAIL), jnp.float32),
            pltpu.SemaphoreType.DMA,
        ],
        input_output_aliases={0: 0},
    )(out_main, u, emb_3_tail, s)
    return out


# speedup vs baseline: 9.9117x; 1.1907x over previous
import jax
import jax.numpy as jnp
from jax import lax
from jax.experimental import pallas as pl
from jax.experimental.pallas import tpu as pltpu
from jax.experimental.pallas import tpu_sc as plsc

B = 1024
M = 50
S = 20
V = 100000
E = 64
EA = 4 * E
L = 16

CH = 8
G = 2
GS = CH * S // G


def _sc_pool_body(story_idx, q_idx, tab,
                  msum, qsum,
                  idx_v, rows_v, out_v,
                  gsem0, gsem1, ssem0, ssem1):
    wid = lax.axis_index("s") * 2 + lax.axis_index("c")
    gsems = (gsem0, gsem1)
    ssems = (ssem0, ssem1)

    def fire(idx_hbm, st, slot):
        pltpu.sync_copy(idx_hbm.at[wid, st], idx_v.at[slot])
        for g in range(G):
            pltpu.async_copy(tab.at[idx_v.at[slot, g]],
                             rows_v.at[slot, pl.ds(g * GS, GS)],
                             gsems[slot])

    def drain_gather(slot):
        for g in range(G):
            pltpu.make_async_copy(tab.at[pl.ds(0, GS)],
                                  rows_v.at[slot, pl.ds(g * GS, GS)],
                                  gsems[slot]).wait()

    def accum_store(out_hbm, seg_base, slot, drain_prev):
        dst = out_hbm.at[pl.ds(seg_base, CH)]
        @pl.when(drain_prev)
        def _():
            pltpu.make_async_copy(out_v.at[slot], dst, ssems[slot]).wait()

        def seg_body(c, carry):
            base = c * S
            for cg in range(EA // L):
                sl = pl.ds(cg * L, L)
                acc = rows_v[slot, base, sl]
                for s2 in range(1, S):
                    acc = acc + rows_v[slot, base + s2, sl]
                out_v[slot, c, sl] = acc
            return carry

        lax.fori_loop(0, CH, seg_body, 0)
        pltpu.async_copy(out_v.at[slot], dst, ssems[slot])

    n_steps = story_idx.shape[1]
    nq_steps = q_idx.shape[1]

    fire(story_idx, 0, 0)

    def outer_body(st0, carry):
        @pl.when(st0 + 1 < n_steps)
        def _():
            fire(story_idx, st0 + 1, 1)

        drain_gather(0)
        accum_store(msum, (wid * n_steps + st0) * CH, 0, st0 >= 2)

        @pl.when(st0 + 2 < n_steps)
        def _():
            fire(story_idx, st0 + 2, 0)

        drain_gather(1)
        accum_store(msum, (wid * n_steps + st0 + 1) * CH, 1, st0 >= 2)
        return carry

    lax.fori_loop(0, n_steps // 2, lambda i, c: outer_body(i * 2, c), 0)

    def qstep_body(st, carry):
        fire(q_idx, st, 0)
        drain_gather(0)
        accum_store(qsum, (wid * nq_steps + st) * CH, 0, st >= 0)
        return carry

    lax.fori_loop(0, nq_steps, qstep_body, 0)

    pltpu.make_async_copy(out_v.at[0], qsum.at[pl.ds(0, CH)], ssems[0]).wait()
    pltpu.make_async_copy(out_v.at[1], msum.at[pl.ds(0, CH)], ssems[1]).wait()


def _concat_body(e0_ref, e1_ref, e2_ref, e3_ref, t_ref):
    t_ref[:, 0 * E:1 * E] = e0_ref[...]
    t_ref[:, 1 * E:2 * E] = e1_ref[...]
    t_ref[:, 2 * E:3 * E] = e2_ref[...]
    t_ref[:, 3 * E:4 * E] = e3_ref[...]


def _hops_body(q_ref, mall_ref, u_ref):
    u = q_ref[...][:, :E]
    mall = mall_ref[...]
    ms = tuple(mall[:, :, i * E:(i + 1) * E] for i in range(4))
    for i in range(3):
        m, c = ms[i], ms[i + 1]
        lg = jnp.sum(m * u[:, None, :], axis=2)
        lg = lg - jnp.max(lg, axis=1, keepdims=True)
        ex = jnp.exp(lg)
        p = ex / jnp.sum(ex, axis=1, keepdims=True)
        u = jnp.sum(c * p[:, :, None], axis=1) + u
    u_ref[...] = u


BT = 256
VT = 5000
NB = B // BT
NV = V // VT


def _sumexp_body(u_ref, e_ref, s_ref):
    v = pl.program_id(0)
    b = pl.program_id(1)
    lg = lax.dot_general(u_ref[...], e_ref[...], (((1,), (1,)), ((), ())),
                         preferred_element_type=jnp.float32)
    part = jnp.sum(jnp.exp(lg), axis=1)[None, :]

    @pl.when(v == 0)
    def _():
        s_ref[pl.ds(b, 1), :] = part

    @pl.when(v != 0)
    def _():
        s_ref[pl.ds(b, 1), :] = s_ref[pl.ds(b, 1), :] + part


VT2 = 6272
NV2 = 15
VTAIL = V - NV2 * VT2


def _make_normexp(nb, row0, aliased):
    def body(*refs):
        if aliased:
            _, u_ref, e_ref, s_ref, o_hbm, buf_ref, sem = refs
        else:
            u_ref, e_ref, s_ref, o_hbm, buf_ref, sem = refs
        v = pl.program_id(0)
        b = pl.program_id(1)
        step = v * nb + b
        slot = step % 2
        lg = lax.dot_general(u_ref[...], e_ref[...],
                             (((1,), (1,)), ((), ())),
                             preferred_element_type=jnp.float32)
        sc = s_ref[pl.ds(b, 1), :]
        res = jnp.exp(lg) * (1.0 / sc[0, :])[:, None]

        dst = o_hbm.at[pl.ds(row0 + b * BT, BT), pl.ds(v * VT2, VT2)]

        @pl.when(step >= 2)
        def _():
            pltpu.make_async_copy(buf_ref.at[slot], dst, sem).wait()

        buf_ref[pl.ds(slot, 1), :, :] = res[None]
        pltpu.make_async_copy(buf_ref.at[slot], dst, sem).start()

        @pl.when(step == NV2 * nb - 1)
        def _():
            pltpu.make_async_copy(buf_ref.at[slot], dst, sem).wait()
            pltpu.make_async_copy(buf_ref.at[1 - slot], dst, sem).wait()

    return body


def _make_tail(row0):
    def body(o_in, u_ref, e_ref, s_ref, o_hbm, buf_ref, sem):
        del o_in
        b = pl.program_id(0)
        lg = lax.dot_general(u_ref[...], e_ref[...],
                             (((1,), (1,)), ((), ())),
                             preferred_element_type=jnp.float32)
        sc = s_ref[pl.ds(b, 1), :]
        buf_ref[...] = jnp.exp(lg) * (1.0 / sc[0, :])[:, None]
        dst = o_hbm.at[pl.ds(row0 + b * BT, BT), pl.ds(NV2 * VT2, VTAIL)]
        pltpu.make_async_copy(buf_ref, dst, sem).start()
        pltpu.make_async_copy(buf_ref, dst, sem).wait()

    return body


NCHK = 2
BH = B // NCHK
NBH = BH // BT


def kernel(story, question, emb_0, emb_1, emb_2, emb_3):
    story = story.astype(jnp.int32)
    question = question.astype(jnp.int32)
    vtc = 2000
    tab = pl.pallas_call(
        _concat_body,
        grid=(V // vtc,),
        in_specs=[pl.BlockSpec((vtc, E), lambda i: (i, 0))] * 4,
        out_specs=pl.BlockSpec((vtc, EA), lambda i: (i, 0)),
        out_shape=jax.ShapeDtypeStruct((V, EA), jnp.float32),
    )(emb_0, emb_1, emb_2, emb_3)

    info = plsc.get_sparse_core_info()
    nw = info.num_cores * info.num_subcores

    n_steps = (BH * M) // (nw * CH)
    nq_steps = BH // (nw * CH)
    story_idx = story.reshape(NCHK, nw, n_steps, G, GS)
    q_idx = question.reshape(NCHK, nw, nq_steps, G, GS)

    mesh = plsc.VectorSubcoreMesh(core_axis_name="c", subcore_axis_name="s")
    sc_pool = pl.kernel(
        _sc_pool_body,
        out_type=(
            jax.ShapeDtypeStruct((BH * M, EA), jnp.float32),
            jax.ShapeDtypeStruct((BH, EA), jnp.float32),
        ),
        mesh=mesh,
        scratch_types=[
            pltpu.VMEM((2, G, GS), jnp.int32),
            pltpu.VMEM((2, CH * S, EA), jnp.float32),
            pltpu.VMEM((2, CH, EA), jnp.float32),
            pltpu.SemaphoreType.DMA,
            pltpu.SemaphoreType.DMA,
            pltpu.SemaphoreType.DMA,
            pltpu.SemaphoreType.DMA,
        ],
    )
    pooled = [sc_pool(story_idx[ci], q_idx[ci], tab) for ci in range(NCHK)]

    bt_h = 128
    emb_3_tail = lax.slice(emb_3, (NV2 * VT2, 0), (V, E))
    out = None
    for ci in range(NCHK):
        msum, qsum = pooled[ci]
        mall = msum.reshape(BH, M, EA)
        u = pl.pallas_call(
            _hops_body,
            grid=(BH // bt_h,),
            in_specs=[
                pl.BlockSpec((bt_h, EA), lambda i: (i, 0)),
                pl.BlockSpec((bt_h, M, EA), lambda i: (i, 0, 0)),
            ],
            out_specs=pl.BlockSpec((bt_h, E), lambda i: (i, 0)),
            out_shape=jax.ShapeDtypeStruct((BH, E), jnp.float32),
        )(qsum, mall)

        s = pl.pallas_call(
            _sumexp_body,
            grid=(NV, NBH),
            in_specs=[
                pl.BlockSpec((BT, E), lambda v, b: (b, 0)),
                pl.BlockSpec((VT, E), lambda v, b: (v, 0)),
            ],
            out_specs=pl.BlockSpec((NBH, BT), lambda v, b: (0, 0)),
            out_shape=jax.ShapeDtypeStruct((NBH, BT), jnp.float32),
        )(u, emb_3)

        aliased = out is not None
        main_in_specs = [
            pl.BlockSpec((BT, E), lambda v, b: (b, 0)),
            pl.BlockSpec((VT2, E), lambda v, b: (v, 0)),
            pl.BlockSpec((NBH, BT), lambda v, b: (0, 0)),
        ]
        main_args = (u, emb_3, s)
        if aliased:
            main_in_specs = [pl.BlockSpec(memory_space=pl.ANY)] + main_in_specs
            main_args = (out,) + main_args
        out = pl.pallas_call(
            _make_normexp(NBH, ci * BH, aliased),
            grid=(NV2, NBH),
            in_specs=main_in_specs,
            out_specs=pl.BlockSpec(memory_space=pl.ANY),
            out_shape=jax.ShapeDtypeStruct((B, V), jnp.float32),
            scratch_shapes=[
                pltpu.VMEM((2, BT, VT2), jnp.float32),
                pltpu.SemaphoreType.DMA,
            ],
            input_output_aliases={0: 0} if aliased else {},
        )(*main_args)

        out = pl.pallas_call(
            _make_tail(ci * BH),
            grid=(NBH,),
            in_specs=[
                pl.BlockSpec(memory_space=pl.ANY),
                pl.BlockSpec((BT, E), lambda b: (b, 0)),
                pl.BlockSpec((VTAIL, E), lambda b: (0, 0)),
                pl.BlockSpec((NBH, BT), lambda b: (0, 0)),
            ],
            out_specs=pl.BlockSpec(memory_space=pl.ANY),
            out_shape=jax.ShapeDtypeStruct((B, V), jnp.float32),
            scratch_shapes=[
                pltpu.VMEM((BT, VTAIL), jnp.float32),
                pltpu.SemaphoreType.DMA,
            ],
            input_output_aliases={0: 0},
        )(out, u, emb_3_tail, s)
    return out
